# baseline (speedup 1.0000x reference)
import functools

import jax
import jax.numpy as jnp
from jax import lax
from jax.experimental import pallas as pl
from jax.experimental.pallas import tpu as pltpu
from jax.experimental.pallas import tpu_sc as plsc

_ROWS_PER_BLOCK = 16384
_VAL_BLOCK = 8192
_IDX_CHUNK = 128


def _sc_scatter_compact(values, idx, cap):
    nv, dim = values.shape
    info = plsc.get_sparse_core_info()
    nc, ns = info.num_cores, info.num_subcores
    nw = nc * ns
    vpw = nv // nw
    nchunks = vpw // _IDX_CHUNK
    idx3 = idx.astype(jnp.int32).reshape(nw, nchunks, _IDX_CHUNK)
    mesh = plsc.VectorSubcoreMesh(core_axis_name="c", subcore_axis_name="s")

    @functools.partial(
        pl.kernel,
        out_type=jax.ShapeDtypeStruct((cap, dim), values.dtype),
        mesh=mesh,
        scratch_types=[
            pltpu.VMEM((nchunks, _IDX_CHUNK), jnp.int32),
            pltpu.VMEM((vpw, dim), values.dtype),
            pltpu.SemaphoreType.DMA,
        ],
    )
    def scatter_kernel(values_hbm, idx_hbm, s_hbm, idx_v, rows_v, sem):
        wid = lax.axis_index("s") * nc + lax.axis_index("c")
        pltpu.sync_copy(idx_hbm.at[wid], idx_v)
        pltpu.sync_copy(values_hbm.at[pl.ds(wid * vpw, vpw)], rows_v)
        copies = [
            pltpu.make_async_copy(
                rows_v.at[pl.ds(j * _IDX_CHUNK, _IDX_CHUNK)],
                s_hbm.at[idx_v.at[j]],
                sem,
            )
            for j in range(nchunks)
        ]
        for cp in copies:
            cp.start()
        for cp in copies:
            cp.wait()

    return scatter_kernel(values, idx3)


def kernel(mem, values, idx):
    cap, dim = mem.shape
    nv = values.shape[0]
    s = _sc_scatter_compact(values, idx, cap)

    r = _ROWS_PER_BLOCK
    first_tail_block = nv // r
    n_tail_blocks = pl.cdiv(cap, r) - first_tail_block

    def tail_body(m_ref, o_ref):
        o_ref[...] = m_ref[...]

    out1 = pl.pallas_call(
        tail_body,
        grid=(n_tail_blocks,),
        in_specs=[
            pl.BlockSpec((r, dim), lambda i: (i + first_tail_block, 0)),
        ],
        out_specs=pl.BlockSpec((r, dim), lambda i: (i + first_tail_block, 0)),
        out_shape=jax.ShapeDtypeStruct((cap, dim), mem.dtype),
    )(mem)

    rv = _VAL_BLOCK

    def values_body(po_ref, s_ref, o_ref):
        del po_ref
        o_ref[...] = s_ref[...]

    return pl.pallas_call(
        values_body,
        grid=(nv // rv,),
        in_specs=[
            pl.BlockSpec(memory_space=pltpu.HBM),
            pl.BlockSpec((rv, dim), lambda i: (i, 0)),
        ],
        out_specs=pl.BlockSpec((rv, dim), lambda i: (i, 0)),
        out_shape=jax.ShapeDtypeStruct((cap, dim), mem.dtype),
        input_output_aliases={0: 0},
    )(out1, s)


# speedup vs baseline: 1.0002x; 1.0002x over previous
import functools

import jax
import jax.numpy as jnp
from jax import lax
from jax.experimental import pallas as pl
from jax.experimental.pallas import tpu as pltpu
from jax.experimental.pallas import tpu_sc as plsc

_ROWS_PER_BLOCK = 16384
_VAL_BLOCK = 8192
_IDX_CHUNK = 128


def _sc_scatter_compact(values, idx, cap):
    nv, dim = values.shape
    info = plsc.get_sparse_core_info()
    nc, ns = info.num_cores, info.num_subcores
    nw = nc * ns
    vpw = nv // nw
    nchunks = vpw // _IDX_CHUNK
    idx3 = idx.astype(jnp.int32).reshape(nw, nchunks, _IDX_CHUNK)
    mesh = plsc.VectorSubcoreMesh(core_axis_name="c", subcore_axis_name="s")

    @functools.partial(
        pl.kernel,
        out_type=jax.ShapeDtypeStruct((cap, dim), values.dtype),
        mesh=mesh,
        scratch_types=[
            pltpu.VMEM((nchunks, _IDX_CHUNK), jnp.int32),
            pltpu.VMEM((vpw, dim), values.dtype),
            pltpu.SemaphoreType.DMA,
        ],
    )
    def scatter_kernel(values_hbm, idx_hbm, s_hbm, idx_v, rows_v, sem):
        wid = lax.axis_index("s") * nc + lax.axis_index("c")
        pltpu.sync_copy(idx_hbm.at[wid], idx_v)
        pltpu.sync_copy(values_hbm.at[pl.ds(wid * vpw, vpw)], rows_v)
        copies = [
            pltpu.make_async_copy(
                rows_v.at[pl.ds(j * _IDX_CHUNK, _IDX_CHUNK)],
                s_hbm.at[idx_v.at[j]],
                sem,
            )
            for j in range(nchunks)
        ]
        for cp in copies:
            cp.start()
        for cp in copies:
            cp.wait()

    return scatter_kernel(values, idx3)


def kernel(mem, values, idx):
    cap, dim = mem.shape
    nv = values.shape[0]

    r = _ROWS_PER_BLOCK
    first_tail_block = nv // r
    n_tail_blocks = pl.cdiv(cap, r) - first_tail_block

    def tail_body(m_ref, o_ref):
        o_ref[...] = m_ref[...]

    out1 = pl.pallas_call(
        tail_body,
        grid=(n_tail_blocks,),
        in_specs=[
            pl.BlockSpec((r, dim), lambda i: (i + first_tail_block, 0)),
        ],
        out_specs=pl.BlockSpec((r, dim), lambda i: (i + first_tail_block, 0)),
        out_shape=jax.ShapeDtypeStruct((cap, dim), mem.dtype),
    )(mem)

    s = _sc_scatter_compact(values, idx, cap)
    rv = _VAL_BLOCK

    def values_body(po_ref, s_ref, o_ref):
        del po_ref
        o_ref[...] = s_ref[...]

    return pl.pallas_call(
        values_body,
        grid=(nv // rv,),
        in_specs=[
            pl.BlockSpec(memory_space=pltpu.HBM),
            pl.BlockSpec((rv, dim), lambda i: (i, 0)),
        ],
        out_specs=pl.BlockSpec((rv, dim), lambda i: (i, 0)),
        out_shape=jax.ShapeDtypeStruct((cap, dim), mem.dtype),
        input_output_aliases={0: 0},
    )(out1, s)


# speedup vs baseline: 1.0487x; 1.0485x over previous
import functools

import jax
import jax.numpy as jnp
from jax import lax
from jax.experimental import pallas as pl
from jax.experimental.pallas import tpu as pltpu
from jax.experimental.pallas import tpu_sc as plsc

_ROWS_PER_BLOCK = 16384
_IDX_CHUNK = 128


def _sc_scatter(values, idx, cap):
    nv, dim = values.shape
    info = plsc.get_sparse_core_info()
    nc, ns = info.num_cores, info.num_subcores
    nw = nc * ns
    vpw = nv // nw
    nchunks = vpw // _IDX_CHUNK
    idx3 = idx.astype(jnp.int32).reshape(nw, nchunks, _IDX_CHUNK)
    mesh = plsc.VectorSubcoreMesh(core_axis_name="c", subcore_axis_name="s")

    @functools.partial(
        pl.kernel,
        out_type=jax.ShapeDtypeStruct((cap, dim), values.dtype),
        mesh=mesh,
        scratch_types=[
            pltpu.VMEM((nchunks, _IDX_CHUNK), jnp.int32),
            pltpu.VMEM((vpw, dim), values.dtype),
            pltpu.SemaphoreType.DMA,
        ],
    )
    def scatter_kernel(values_hbm, idx_hbm, out_hbm, idx_v, rows_v, sem):
        wid = lax.axis_index("s") * nc + lax.axis_index("c")
        pltpu.sync_copy(idx_hbm.at[wid], idx_v)
        pltpu.sync_copy(values_hbm.at[pl.ds(wid * vpw, vpw)], rows_v)
        copies = [
            pltpu.make_async_copy(
                rows_v.at[pl.ds(j * _IDX_CHUNK, _IDX_CHUNK)],
                out_hbm.at[idx_v.at[j]],
                sem,
            )
            for j in range(nchunks)
        ]
        for cp in copies:
            cp.start()
        for cp in copies:
            cp.wait()

    return scatter_kernel(values, idx3)


def kernel(mem, values, idx):
    cap, dim = mem.shape
    nv = values.shape[0]
    partial_out = _sc_scatter(values, idx, cap)

    r = _ROWS_PER_BLOCK
    first_tail_block = nv // r
    n_tail_blocks = pl.cdiv(cap, r) - first_tail_block

    def tail_body(po_ref, m_ref, o_ref):
        del po_ref
        o_ref[...] = m_ref[...]

    return pl.pallas_call(
        tail_body,
        grid=(n_tail_blocks,),
        in_specs=[
            pl.BlockSpec(memory_space=pltpu.HBM),
            pl.BlockSpec((r, dim), lambda i: (i + first_tail_block, 0)),
        ],
        out_specs=pl.BlockSpec((r, dim), lambda i: (i + first_tail_block, 0)),
        out_shape=jax.ShapeDtypeStruct((cap, dim), mem.dtype),
        input_output_aliases={0: 0},
    )(partial_out, mem)
